# baseline (speedup 1.0000x reference)
import functools
import math

import jax
import jax.numpy as jnp
from jax.experimental import pallas as pl
from jax.experimental.pallas import tpu as pltpu

_INV_SQRT2 = 1.0 / math.sqrt(2.0)
_VMEM_LIMIT = 64 * 1024 * 1024


def _gelu(x):
    return 0.5 * x * (1.0 + jax.lax.erf(x * _INV_SQRT2))


def _convmixer_kernel(p_ref, pe_w_ref, pe_aux_ref,
                      dw_w_ref, dw_aux_ref, pw_w_ref, pw_aux_ref,
                      o_ref, ybuf_ref, s_ref, *, HP, WP, K, PAD, L):
    HW = HP * WP
    HH = WP // 2
    D = o_ref.shape[-1]

    def write_shifts(p, row_bf16):
        zc = jnp.zeros((PAD, D), jnp.bfloat16)
        padded = jnp.concatenate([zc, row_bf16, zc], axis=0)
        for j in range(K):
            s_ref[j, p] = padded[j:j + WP, :]

    zrow = jnp.zeros((WP, D), jnp.bfloat16)
    for j in range(K):
        for r in range(PAD):
            s_ref[j, r] = zrow
            s_ref[j, HP + PAD + r] = zrow

    feat = jnp.dot(p_ref[0], pe_w_ref[...], preferred_element_type=jnp.float32)
    a = pe_aux_ref[...]
    feat = _gelu(feat + a[0]) * a[1] + a[2]
    for r in range(HP):
        write_shifts(r + PAD, feat[r * WP:(r + 1) * WP].astype(jnp.bfloat16))

    def layer(l, _):
        daux = [dw_aux_ref[l, i] for i in range(3)]

        def half_epilogue(g, aux, x0):
            b, s, t = aux
            xg = g + b
            e = jax.lax.erf(xg * _INV_SQRT2)
            p = xg + xg * e
            return (p * s + (t + x0)).astype(jnp.bfloat16)

        def row(h, _):
            accA = accB = None
            for i in range(K):
                for j in range(K):
                    wt = dw_w_ref[l, i * K + j]
                    if j == 0 and i % 3 == 0:
                        gA = s_ref[0, h + i, 0:HH, :] * wt
                        gB = s_ref[0, h + i, HH:WP, :] * wt
                    else:
                        gA = gA + s_ref[j, h + i, 0:HH, :] * wt
                        gB = gB + s_ref[j, h + i, HH:WP, :] * wt
                if i % 3 == 2:
                    if accA is None:
                        accA = gA.astype(jnp.float32)
                        accB = gB.astype(jnp.float32)
                    else:
                        accA = accA + gA.astype(jnp.float32)
                        accB = accB + gB.astype(jnp.float32)
            yA = half_epilogue(accA, daux,
                               s_ref[PAD, h + PAD, 0:HH, :].astype(jnp.float32))
            yB = half_epilogue(accB, daux,
                               s_ref[PAD, h + PAD, HH:WP, :].astype(jnp.float32))
            ybuf_ref[pl.ds(h * WP, HH), :] = yA
            ybuf_ref[pl.ds(h * WP + HH, HH), :] = yB
            return 0

        jax.lax.fori_loop(0, HP, row, 0)

        z = jnp.dot(ybuf_ref[...], pw_w_ref[l],
                    preferred_element_type=jnp.float32)
        paux = [pw_aux_ref[l, i] for i in range(3)]
        pzero = jnp.zeros_like(paux[0])
        for r in range(HP):
            zoA = half_epilogue(z[r * WP:r * WP + HH], paux, pzero)
            zoB = half_epilogue(z[r * WP + HH:(r + 1) * WP], paux, pzero)
            write_shifts(r + PAD, jnp.concatenate([zoA, zoB], axis=0))
        return 0

    jax.lax.fori_loop(0, L, layer, 0)

    interior = s_ref[PAD, pl.ds(PAD, HP), :, :].astype(jnp.float32)
    o_ref[0, 0, :] = jnp.mean(interior, axis=(0, 1))


def _convmixer_fused(patches, pe_w, pe_aux, dw_w, dw_aux, pw_w, pw_aux,
                     *, HP, WP, K):
    n = patches.shape[0]
    cpp = patches.shape[2]
    d = pe_w.shape[1]
    L = dw_w.shape[0]
    pad = K // 2
    hw = HP * WP

    kern = functools.partial(_convmixer_kernel, HP=HP, WP=WP, K=K, PAD=pad, L=L)
    out = pl.pallas_call(
        kern,
        out_shape=jax.ShapeDtypeStruct((n, 1, d), jnp.float32),
        grid_spec=pltpu.PrefetchScalarGridSpec(
            num_scalar_prefetch=0,
            grid=(n,),
            in_specs=[
                pl.BlockSpec((1, hw, cpp), lambda b: (b, 0, 0)),
                pl.BlockSpec((cpp, d), lambda b: (0, 0)),
                pl.BlockSpec((3, d), lambda b: (0, 0)),
                pl.BlockSpec((L, K * K, 16, d), lambda b: (0, 0, 0, 0)),
                pl.BlockSpec((L, 3, 16, d), lambda b: (0, 0, 0, 0)),
                pl.BlockSpec((L, d, d), lambda b: (0, 0, 0)),
                pl.BlockSpec((L, 3, 16, d), lambda b: (0, 0, 0, 0)),
            ],
            out_specs=pl.BlockSpec((1, 1, d), lambda b: (b, 0, 0)),
            scratch_shapes=[
                pltpu.VMEM((hw, d), jnp.bfloat16),
                pltpu.VMEM((K, HP + 2 * pad, WP, d), jnp.bfloat16),
            ],
        ),
        compiler_params=pltpu.CompilerParams(
            dimension_semantics=("parallel",),
            vmem_limit_bytes=_VMEM_LIMIT),
        cost_estimate=pl.CostEstimate(
            flops=n * hw * d * (2 * cpp + L * (2 * K * K + 2 * d)),
            transcendentals=n * hw * d * (1 + 2 * L),
            bytes_accessed=n * hw * cpp * 2 + L * d * d * 2 + n * d * 4),
    )(patches, pe_w, pe_aux, dw_w, dw_aux, pw_w, pw_aux)
    return out.reshape(n, d)


def kernel(x, pe_w, pe_b, pe_scale, pe_shift, l0_dw_w, l0_dw_b, l0_dw_scale, l0_dw_shift, l0_pw_w, l0_pw_b, l0_pw_scale, l0_pw_shift, l1_dw_w, l1_dw_b, l1_dw_scale, l1_dw_shift, l1_pw_w, l1_pw_b, l1_pw_scale, l1_pw_shift, l2_dw_w, l2_dw_b, l2_dw_scale, l2_dw_shift, l2_pw_w, l2_pw_b, l2_pw_scale, l2_pw_shift, l3_dw_w, l3_dw_b, l3_dw_scale, l3_dw_shift, l3_pw_w, l3_pw_b, l3_pw_scale, l3_pw_shift, l4_dw_w, l4_dw_b, l4_dw_scale, l4_dw_shift, l4_pw_w, l4_pw_b, l4_pw_scale, l4_pw_shift, l5_dw_w, l5_dw_b, l5_dw_scale, l5_dw_shift, l5_pw_w, l5_pw_b, l5_pw_scale, l5_pw_shift, l6_dw_w, l6_dw_b, l6_dw_scale, l6_dw_shift, l6_pw_w, l6_pw_b, l6_pw_scale, l6_pw_shift, l7_dw_w, l7_dw_b, l7_dw_scale, l7_dw_shift, l7_pw_w, l7_pw_b, l7_pw_scale, l7_pw_shift):
    n, c, h, w = x.shape
    p = 7
    hp, wp = h // p, w // p
    d = pe_w.shape[1]
    kk = int(round(math.sqrt(l0_dw_w.shape[0])))

    patches = (x.reshape(n, c, hp, p, wp, p)
                .transpose(0, 2, 4, 1, 3, 5)
                .reshape(n, hp * wp, c * p * p)
                .astype(jnp.bfloat16))

    layers = [
        (l0_dw_w, l0_dw_b, l0_dw_scale, l0_dw_shift, l0_pw_w, l0_pw_b, l0_pw_scale, l0_pw_shift),
        (l1_dw_w, l1_dw_b, l1_dw_scale, l1_dw_shift, l1_pw_w, l1_pw_b, l1_pw_scale, l1_pw_shift),
        (l2_dw_w, l2_dw_b, l2_dw_scale, l2_dw_shift, l2_pw_w, l2_pw_b, l2_pw_scale, l2_pw_shift),
        (l3_dw_w, l3_dw_b, l3_dw_scale, l3_dw_shift, l3_pw_w, l3_pw_b, l3_pw_scale, l3_pw_shift),
        (l4_dw_w, l4_dw_b, l4_dw_scale, l4_dw_shift, l4_pw_w, l4_pw_b, l4_pw_scale, l4_pw_shift),
        (l5_dw_w, l5_dw_b, l5_dw_scale, l5_dw_shift, l5_pw_w, l5_pw_b, l5_pw_scale, l5_pw_shift),
        (l6_dw_w, l6_dw_b, l6_dw_scale, l6_dw_shift, l6_pw_w, l6_pw_b, l6_pw_scale, l6_pw_shift),
        (l7_dw_w, l7_dw_b, l7_dw_scale, l7_dw_shift, l7_pw_w, l7_pw_b, l7_pw_scale, l7_pw_shift),
    ]
    pe_aux = jnp.stack([pe_b, pe_scale, pe_shift])
    dw_w_all = jnp.stack([lw[0] for lw in layers]).astype(jnp.bfloat16)
    dw_w_rep = jnp.broadcast_to(dw_w_all[:, :, None, :],
                                (len(layers), dw_w_all.shape[1], 16, d))
    dw_aux = jnp.stack([jnp.stack([lw[1], 0.5 * lw[2], lw[3]]) for lw in layers])
    dw_aux = jnp.broadcast_to(dw_aux[:, :, None, :], (len(layers), 3, 16, d))
    pw_w_all = jnp.stack([lw[4] for lw in layers]).astype(jnp.bfloat16)
    pw_aux = jnp.stack([jnp.stack([lw[5], 0.5 * lw[6], lw[7]]) for lw in layers])
    pw_aux = jnp.broadcast_to(pw_aux[:, :, None, :], (len(layers), 3, 16, d))

    return _convmixer_fused(patches, pe_w.astype(jnp.bfloat16), pe_aux,
                            dw_w_rep, dw_aux, pw_w_all, pw_aux,
                            HP=hp, WP=wp, K=kk)
